# baseline (speedup 1.0000x reference)
import functools

import jax
import jax.numpy as jnp
from jax import lax
from jax.experimental import pallas as pl
from jax.experimental.pallas import tpu as pltpu
from jax.experimental.pallas import tpu_sc as plsc

_NEG_SLOPE = 0.01
_TW = 144
_CHUNK = 128
_EPAD = 327680
_NPAD = 10240

_BCAST_DN = lax.GatherDimensionNumbers(
    offset_dims=(), collapsed_slice_dims=(0,), start_index_map=(0,))


def _lane_bcast(vec16, lane):
    idx = jnp.full((16, 1), lane, jnp.int32)
    return lax.gather(vec16, idx, _BCAST_DN, slice_sizes=(1,),
                      mode=lax.GatherScatterMode.PROMISE_IN_BOUNDS)


def _lstm_body(wg0, wihT0, whhT0, b0, wg1, wihT1, whhT1, b1,
               out0, out1, zx0, zx1):
    T, H = wg0.shape

    def one_pass(src0, src1):
        zx0[...] = jnp.dot(src0, wihT0[...],
                           preferred_element_type=jnp.float32) + b0[...]
        zx1[...] = jnp.dot(src1, wihT1[...],
                           preferred_element_type=jnp.float32) + b1[...]

        def step(t, carry):
            h0, c0, h1, c1 = carry

            def cell(zx, h, c, whhT, out):
                z = zx[pl.ds(t, 1), :] + jnp.dot(
                    h, whhT[...], preferred_element_type=jnp.float32)
                i = jax.nn.sigmoid(z[:, 0:H])
                f = jax.nn.sigmoid(z[:, H:2 * H])
                g = jnp.tanh(z[:, 2 * H:3 * H])
                o = jax.nn.sigmoid(z[:, 3 * H:4 * H])
                c = f * c + i * g
                h = o * jnp.tanh(c)
                out[pl.ds(t, 1), :] = h
                return h, c

            h0, c0 = cell(zx0, h0, c0, whhT0, out0)
            h1, c1 = cell(zx1, h1, c1, whhT1, out1)
            return h0, c0, h1, c1

        z = jnp.zeros((1, H), jnp.float32)
        lax.fori_loop(0, T, step, (z, z, z, z))

    one_pass(wg0[...], wg1[...])
    one_pass(out0[...], out1[...])


def _lstm_pallas(Wgat0, Wih0, Whh0, bih0, bhh0, Wgat1, Wih1, Whh1, bih1, bhh1):
    b0 = (bih0 + bhh0).reshape(1, -1)
    b1 = (bih1 + bhh1).reshape(1, -1)
    return pl.pallas_call(
        _lstm_body,
        out_shape=[jax.ShapeDtypeStruct(Wgat0.shape, jnp.float32)] * 2,
        scratch_shapes=[pltpu.VMEM((128, 512), jnp.float32)] * 2,
    )(Wgat0, Wih0.T, Whh0.T, b0, Wgat1, Wih1.T, Whh1.T, b1)


def _ft_ps_body(x, W, a_s, ft, ps, mx):
    xv = x[...]
    ft[...] = jnp.dot(xv, W[...], preferred_element_type=jnp.float32)
    p = jnp.dot(xv, a_s[...], preferred_element_type=jnp.float32)
    ps[...] = p
    mx[...] = jnp.full((1, 1), jnp.max(p))


def _ft_ps_pallas(x, W, a_s):
    return pl.pallas_call(
        _ft_ps_body,
        out_shape=[jax.ShapeDtypeStruct(x.shape, jnp.float32),
                   jax.ShapeDtypeStruct((x.shape[0], 1), jnp.float32),
                   jax.ShapeDtypeStruct((1, 1), jnp.float32)],
    )(x, W, a_s)


def _norm_ft_ps_body(am, bm, da, db, W, a_s, ft, ps, mx):
    x = (am[...] + bm[...]) / (da[...] + db[...] + 1e-16)
    x = jnp.maximum(x, x * _NEG_SLOPE)
    ft[...] = jnp.dot(x, W[...], preferred_element_type=jnp.float32)
    p = jnp.dot(x, a_s[...], preferred_element_type=jnp.float32)
    ps[...] = p
    mx[...] = jnp.full((1, 1), jnp.max(p))


def _norm_ft_ps_pallas(acc2, den2, W, a_s):
    n = acc2.shape[0] // 2
    am, bm = acc2[:n], acc2[n:]
    da, db = den2[:n].reshape(n, 1), den2[n:].reshape(n, 1)
    return pl.pallas_call(
        _norm_ft_ps_body,
        out_shape=[jax.ShapeDtypeStruct((n, 128), jnp.float32),
                   jax.ShapeDtypeStruct((n, 1), jnp.float32),
                   jax.ShapeDtypeStruct((1, 1), jnp.float32)],
    )(am, bm, da, db, W, a_s)


def _norm_body(am, bm, da, db, out):
    x = (am[...] + bm[...]) / (da[...] + db[...] + 1e-16)
    out[...] = jnp.maximum(x, x * _NEG_SLOPE)


def _norm_pallas(acc2, den2):
    n = acc2.shape[0] // 2
    am, bm = acc2[:n], acc2[n:]
    da, db = den2[:n].reshape(n, 1), den2[n:].reshape(n, 1)
    return pl.pallas_call(
        _norm_body,
        out_shape=jax.ShapeDtypeStruct((n, 128), jnp.float32),
    )(am, bm, da, db)


def _pe_body(ea, a_e, pe, mx, mscr):
    i = pl.program_id(0)
    p = jnp.dot(ea[...], a_e[...], preferred_element_type=jnp.float32)
    pe[...] = p
    m = jnp.max(p)

    @pl.when(i == 0)
    def _():
        mscr[0, 0] = m

    @pl.when(i > 0)
    def _():
        mscr[0, 0] = jnp.maximum(mscr[0, 0], m)

    @pl.when(i == pl.num_programs(0) - 1)
    def _():
        mx[...] = jnp.full((1, 1), mscr[0, 0])


def _pe_pallas(eattr, a_e):
    E = eattr.shape[0]
    nb = 32
    blk = E // nb
    return pl.pallas_call(
        _pe_body,
        grid=(nb,),
        in_specs=[pl.BlockSpec((blk, eattr.shape[1]), lambda i: (i, 0)),
                  pl.BlockSpec((eattr.shape[1], 1), lambda i: (0, 0))],
        out_specs=[pl.BlockSpec((blk, 1), lambda i: (i, 0)),
                   pl.BlockSpec((1, 1), lambda i: (0, 0))],
        out_shape=[jax.ShapeDtypeStruct((E, 1), jnp.float32),
                   jax.ShapeDtypeStruct((1, 1), jnp.float32)],
        scratch_shapes=[pltpu.SMEM((1, 1), jnp.float32)],
    )(eattr, a_e)


def _edge_body(ft, src, dst, pe, ps, mps, mpe, acc2, den2,
               acc, den,
               srcv0, dstv0, pev0, psg0, wbuf0, rows0,
               srcv1, dstv1, pev1, psg1, wbuf1, rows1,
               zbuf, zdb, shv_v,
               lsp0, lsp1, ld0, ld1, gr0, gr1, gp0, gp1, sr0, sr1, sd0, sd1):
    info = plsc.get_sparse_core_info()
    nc, ns = info.num_cores, info.num_subcores
    cid = lax.axis_index("c")
    sid = lax.axis_index("s")
    wid = sid * nc + cid
    E = src.shape[0]
    n_rows = acc2.shape[0] // 2
    rows_per_sub = n_rows // ns
    edges_per_w = E // (nc * ns)
    n_chunks = edges_per_w // _CHUNK
    e0 = wid * edges_per_w

    srcv = (srcv0, srcv1)
    dstv = (dstv0, dstv1)
    pev = (pev0, pev1)
    psg = (psg0, psg1)
    wbuf = (wbuf0, wbuf1)
    rows = (rows0, rows1)
    lsp = (lsp0, lsp1)
    ld = (ld0, ld1)
    gr = (gr0, gr1)
    gp = (gp0, gp1)
    sr = (sr0, sr1)
    sd = (sd0, sd1)

    zv = jnp.zeros((16,), jnp.float32)

    def zrow(r, _):
        for j in range(128 // 16):
            zbuf[r, pl.ds(16 * j, 16)] = zv
        return 0

    lax.fori_loop(0, zbuf.shape[0], zrow, 0)

    def zdrow(r, _):
        zdb[pl.ds(16 * r, 16)] = zv
        return 0

    lax.fori_loop(0, rows_per_sub // 16, zdrow, 0)
    zchunk = zbuf.shape[0]
    for i in range(rows_per_sub // zchunk):
        pltpu.sync_copy(zbuf, acc.at[pl.ds(sid * rows_per_sub
                                           + i * zchunk, zchunk)])
    pltpu.sync_copy(zdb, den.at[pl.ds(sid * rows_per_sub, rows_per_sub)])
    plsc.subcore_barrier()

    pltpu.sync_copy(mps, shv_v)
    shv = shv_v[...]
    pltpu.sync_copy(mpe, shv_v)
    shv = shv + shv_v[...]

    def lin_sp(g, b):
        base = e0 + g * _CHUNK
        pltpu.async_copy(src.at[pl.ds(base, _CHUNK)], srcv[b], lsp[b])
        pltpu.async_copy(pe.at[pl.ds(base, _CHUNK)], pev[b], lsp[b])

    def wait_sp(b):
        pltpu.make_async_copy(src.at[pl.ds(0, _CHUNK)], srcv[b], lsp[b]).wait()
        pltpu.make_async_copy(pe.at[pl.ds(0, _CHUNK)], pev[b], lsp[b]).wait()

    def lin_d(g, b):
        base = e0 + g * _CHUNK
        pltpu.async_copy(dst.at[pl.ds(base, _CHUNK)], dstv[b], ld[b])

    def wait_d(b):
        pltpu.make_async_copy(dst.at[pl.ds(0, _CHUNK)], dstv[b], ld[b]).wait()

    def gat(b):
        pltpu.async_copy(ft.at[srcv[b]], rows[b], gr[b])
        pltpu.async_copy(ps.at[srcv[b]], psg[b], gp[b])

    def wait_gat(b):
        pltpu.make_async_copy(ft.at[srcv[b]], rows[b], gr[b]).wait()
        pltpu.make_async_copy(ps.at[srcv[b]], psg[b], gp[b]).wait()

    def scat(b):
        pltpu.async_copy(rows[b], acc.at[dstv[b]], sr[b], add=True)
        pltpu.async_copy(wbuf[b], den.at[dstv[b]], sd[b], add=True)

    def wait_scat(b):
        pltpu.make_async_copy(rows[b], acc.at[dstv[b]], sr[b]).wait()
        pltpu.make_async_copy(wbuf[b], den.at[dstv[b]], sd[b]).wait()

    def scale(b):
        for gi in range(_CHUNK // 16):
            sl16 = pl.ds(16 * gi, 16)
            wv = jnp.exp(pev[b][sl16] + psg[b][sl16] - shv)
            wbuf[b][sl16] = wv
            for r in range(16):
                bc = _lane_bcast(wv, r)
                row = gi * 16 + r
                for j in range(128 // 16):
                    sl = pl.ds(16 * j, 16)
                    rows[b][row, sl] = rows[b][row, sl] * bc

    def step(g, b, first, last2):
        nb = 1 - b
        wait_gat(b)
        scale(b)
        wait_d(b)
        scat(b)
        wait_sp(nb)
        if not first:
            wait_scat(nb)
        lin_d(g + 1, nb)
        gat(nb)
        if last2 is None:
            @pl.when(g <= n_chunks - 3)
            def _():
                lin_sp(g + 2, b)
        elif not last2:
            lin_sp(g + 2, b)

    lin_sp(0, 0)
    lin_d(0, 0)
    lin_sp(1, 1)
    wait_sp(0)
    gat(0)
    step(0, 0, True, False)

    def pair(p, _):
        g = 2 * p + 1
        step(g, 1, False, None)
        step(g + 1, 0, False, None)
        return 0

    lax.fori_loop(0, (n_chunks - 4) // 2, pair, 0)
    step(n_chunks - 3, 1, False, False)
    step(n_chunks - 2, 0, False, True)

    wait_gat(1)
    scale(1)
    wait_d(1)
    scat(1)
    wait_scat(0)
    wait_scat(1)
    plsc.subcore_barrier()

    pltpu.sync_copy(
        acc.at[pl.ds(sid * rows_per_sub, rows_per_sub)],
        acc2.at[pl.ds(cid * n_rows + sid * rows_per_sub, rows_per_sub)])
    pltpu.sync_copy(
        den.at[pl.ds(sid * rows_per_sub, rows_per_sub)],
        den2.at[pl.ds(cid * n_rows + sid * rows_per_sub, rows_per_sub)])


def _edge_pallas(ft, src, dst, pe, ps, mps16, mpe16):
    n = ft.shape[0]
    mesh = plsc.VectorSubcoreMesh(core_axis_name="c", subcore_axis_name="s")
    return pl.kernel(
        _edge_body,
        out_type=[jax.ShapeDtypeStruct((2 * n, 128), jnp.float32),
                  jax.ShapeDtypeStruct((2 * n,), jnp.float32)],
        mesh=mesh,
        scratch_types=[
            pltpu.VMEM_SHARED((_NPAD, 128), jnp.float32),
            pltpu.VMEM_SHARED((_NPAD,), jnp.float32),
            pltpu.VMEM((_CHUNK,), jnp.int32),
            pltpu.VMEM((_CHUNK,), jnp.int32),
            pltpu.VMEM((_CHUNK,), jnp.float32),
            pltpu.VMEM((_CHUNK,), jnp.float32),
            pltpu.VMEM((_CHUNK,), jnp.float32),
            pltpu.VMEM((_CHUNK, 128), jnp.float32),
            pltpu.VMEM((_CHUNK,), jnp.int32),
            pltpu.VMEM((_CHUNK,), jnp.int32),
            pltpu.VMEM((_CHUNK,), jnp.float32),
            pltpu.VMEM((_CHUNK,), jnp.float32),
            pltpu.VMEM((_CHUNK,), jnp.float32),
            pltpu.VMEM((_CHUNK, 128), jnp.float32),
            pltpu.VMEM((32, 128), jnp.float32),
            pltpu.VMEM((640,), jnp.float32),
            pltpu.VMEM((16,), jnp.float32),
        ] + [pltpu.SemaphoreType.DMA] * 12,
    )(ft, src, dst, pe, ps, mps16, mpe16)


def kernel(x0, x1, edge_index0, edge_index1, eattr0, eattr1, Wgat0, Wgat1,
           a0, a1, Wih0, Whh0, bih0, bhh0, Wih1, Whh1, bih1, bhh1):
    N, F = x1.shape
    DE = eattr1.shape[1]
    epad = _EPAD - edge_index1.shape[1]
    src = jnp.pad(edge_index1[0], (0, epad))
    dst = jnp.pad(edge_index1[1], (0, epad), constant_values=N + 100)
    W0, W1 = _lstm_pallas(Wgat0, Wih0, Whh0, bih0, bhh0,
                          Wgat1, Wih1, Whh1, bih1, bhh1)
    xp = jnp.pad(x1, ((0, _NPAD - N), (0, 0)))
    pes, mpes, a_ss = [], [], []
    for a in (a0, a1):
        a_e = a[0, F:F + DE].reshape(DE, 1)
        pe, mpe = _pe_pallas(eattr1, a_e)
        pes.append(jnp.pad(pe.reshape(-1), (0, epad)))
        mpes.append(jnp.broadcast_to(mpe.reshape(1), (16,)))
        a_ss.append(a[0, :F].reshape(F, 1))
    ft, ps, mps = _ft_ps_pallas(xp, W0, a_ss[0])
    mps16 = jnp.broadcast_to(mps.reshape(1), (16,))
    acc2, den2 = _edge_pallas(ft, src, dst, pes[0], ps.reshape(-1),
                              mps16, mpes[0])
    ft, ps, mps = _norm_ft_ps_pallas(acc2, den2, W1, a_ss[1])
    mps16 = jnp.broadcast_to(mps.reshape(1), (16,))
    acc2, den2 = _edge_pallas(ft, src, dst, pes[1], ps.reshape(-1),
                              mps16, mpes[1])
    return _norm_pallas(acc2, den2)[:N]


# speedup vs baseline: 1.7338x; 1.7338x over previous
import functools

import jax
import jax.numpy as jnp
from jax import lax
from jax.experimental import pallas as pl
from jax.experimental.pallas import tpu as pltpu
from jax.experimental.pallas import tpu_sc as plsc

_NEG_SLOPE = 0.01
_TW = 144
_CHUNK = 128
_EPAD = 327680
_NPAD = 10240

_BCAST_DN = lax.GatherDimensionNumbers(
    offset_dims=(), collapsed_slice_dims=(0,), start_index_map=(0,))


def _lane_bcast(vec16, lane):
    idx = jnp.full((16, 1), lane, jnp.int32)
    return lax.gather(vec16, idx, _BCAST_DN, slice_sizes=(1,),
                      mode=lax.GatherScatterMode.PROMISE_IN_BOUNDS)


def _lstm_body(wg0, wihT0, whhT0, b0, wg1, wihT1, whhT1, b1,
               out0, out1, zx0, zx1):
    T, H = wg0.shape

    def one_pass(src0, src1):
        zx0[...] = jnp.dot(src0, wihT0[...],
                           preferred_element_type=jnp.float32) + b0[...]
        zx1[...] = jnp.dot(src1, wihT1[...],
                           preferred_element_type=jnp.float32) + b1[...]

        def step(t, carry):
            h0, c0, h1, c1 = carry

            def cell(zx, h, c, whhT, out):
                z = zx[pl.ds(t, 1), :] + jnp.dot(
                    h, whhT[...], preferred_element_type=jnp.float32)
                i = jax.nn.sigmoid(z[:, 0:H])
                f = jax.nn.sigmoid(z[:, H:2 * H])
                g = jnp.tanh(z[:, 2 * H:3 * H])
                o = jax.nn.sigmoid(z[:, 3 * H:4 * H])
                c = f * c + i * g
                h = o * jnp.tanh(c)
                out[pl.ds(t, 1), :] = h
                return h, c

            h0, c0 = cell(zx0, h0, c0, whhT0, out0)
            h1, c1 = cell(zx1, h1, c1, whhT1, out1)
            return h0, c0, h1, c1

        z = jnp.zeros((1, H), jnp.float32)
        lax.fori_loop(0, T, step, (z, z, z, z))

    one_pass(wg0[...], wg1[...])
    one_pass(out0[...], out1[...])


def _lstm_pallas(Wgat0, Wih0, Whh0, bih0, bhh0, Wgat1, Wih1, Whh1, bih1, bhh1):
    b0 = (bih0 + bhh0).reshape(1, -1)
    b1 = (bih1 + bhh1).reshape(1, -1)
    return pl.pallas_call(
        _lstm_body,
        out_shape=[jax.ShapeDtypeStruct(Wgat0.shape, jnp.float32)] * 2,
        scratch_shapes=[pltpu.VMEM((128, 512), jnp.float32)] * 2,
    )(Wgat0, Wih0.T, Whh0.T, b0, Wgat1, Wih1.T, Whh1.T, b1)


def _ft_ps_body(x, W, a_s, ft, ps, mx):
    xv = x[...]
    ft[...] = jnp.dot(xv, W[...], preferred_element_type=jnp.float32)
    p = jnp.dot(xv, a_s[...], preferred_element_type=jnp.float32)
    ps[...] = p
    mx[...] = jnp.full((1, 1), jnp.max(p))


def _ft_ps_pallas(x, W, a_s):
    return pl.pallas_call(
        _ft_ps_body,
        out_shape=[jax.ShapeDtypeStruct(x.shape, jnp.float32),
                   jax.ShapeDtypeStruct((x.shape[0], 1), jnp.float32),
                   jax.ShapeDtypeStruct((1, 1), jnp.float32)],
    )(x, W, a_s)


def _norm_ft_ps_body(am, bm, da, db, W, a_s, ft, ps, mx):
    x = (am[...] + bm[...]) / (da[...] + db[...] + 1e-16)
    x = jnp.maximum(x, x * _NEG_SLOPE)
    ft[...] = jnp.dot(x, W[...], preferred_element_type=jnp.float32)
    p = jnp.dot(x, a_s[...], preferred_element_type=jnp.float32)
    ps[...] = p
    mx[...] = jnp.full((1, 1), jnp.max(p))


def _norm_ft_ps_pallas(acc2, den2, W, a_s):
    n = acc2.shape[0] // 2
    am, bm = acc2[:n], acc2[n:]
    da, db = den2[:n].reshape(n, 1), den2[n:].reshape(n, 1)
    return pl.pallas_call(
        _norm_ft_ps_body,
        out_shape=[jax.ShapeDtypeStruct((n, 128), jnp.float32),
                   jax.ShapeDtypeStruct((n, 1), jnp.float32),
                   jax.ShapeDtypeStruct((1, 1), jnp.float32)],
    )(am, bm, da, db, W, a_s)


def _norm_body(am, bm, da, db, out):
    x = (am[...] + bm[...]) / (da[...] + db[...] + 1e-16)
    out[...] = jnp.maximum(x, x * _NEG_SLOPE)


def _norm_pallas(acc2, den2):
    n = acc2.shape[0] // 2
    am, bm = acc2[:n], acc2[n:]
    da, db = den2[:n].reshape(n, 1), den2[n:].reshape(n, 1)
    return pl.pallas_call(
        _norm_body,
        out_shape=jax.ShapeDtypeStruct((n, 128), jnp.float32),
    )(am, bm, da, db)


def _pe_body(ea, a_e, pe, mx, mscr):
    i = pl.program_id(0)
    p = jnp.dot(ea[...], a_e[...], preferred_element_type=jnp.float32)
    pe[...] = p
    m = jnp.max(p)

    @pl.when(i == 0)
    def _():
        mscr[0, 0] = m

    @pl.when(i > 0)
    def _():
        mscr[0, 0] = jnp.maximum(mscr[0, 0], m)

    @pl.when(i == pl.num_programs(0) - 1)
    def _():
        mx[...] = jnp.full((1, 1), mscr[0, 0])


def _pe_pallas(eattr, a_e):
    E = eattr.shape[0]
    nb = 32
    blk = E // nb
    return pl.pallas_call(
        _pe_body,
        grid=(nb,),
        in_specs=[pl.BlockSpec((blk, eattr.shape[1]), lambda i: (i, 0)),
                  pl.BlockSpec((eattr.shape[1], 1), lambda i: (0, 0))],
        out_specs=[pl.BlockSpec((blk, 1), lambda i: (i, 0)),
                   pl.BlockSpec((1, 1), lambda i: (0, 0))],
        out_shape=[jax.ShapeDtypeStruct((E, 1), jnp.float32),
                   jax.ShapeDtypeStruct((1, 1), jnp.float32)],
        scratch_shapes=[pltpu.SMEM((1, 1), jnp.float32)],
    )(eattr, a_e)


def _edge_body(ft, src, dst, pe, ps, mps, mpe, acc2, den2,
               acc, den,
               srcv0, dstv0, pev0, psg0, wbuf0, rows0,
               srcv1, dstv1, pev1, psg1, wbuf1, rows1,
               zbuf, zdb, shv_v,
               lsp0, lsp1, ld0, ld1, gr0, gr1, gp0, gp1, sr0, sr1, sd0, sd1):
    info = plsc.get_sparse_core_info()
    nc, ns = info.num_cores, info.num_subcores
    cid = lax.axis_index("c")
    sid = lax.axis_index("s")
    wid = sid * nc + cid
    E = src.shape[0]
    n_rows = acc2.shape[0] // 2
    rows_per_sub = n_rows // ns
    edges_per_w = E // (nc * ns)
    n_chunks = edges_per_w // _CHUNK
    e0 = wid * edges_per_w

    srcv = (srcv0, srcv1)
    dstv = (dstv0, dstv1)
    pev = (pev0, pev1)
    psg = (psg0, psg1)
    wbuf = (wbuf0, wbuf1)
    rows = (rows0, rows1)
    lsp = (lsp0, lsp1)
    ld = (ld0, ld1)
    gr = (gr0, gr1)
    gp = (gp0, gp1)
    sr = (sr0, sr1)
    sd = (sd0, sd1)

    zv = jnp.zeros((16,), jnp.float32)

    def zrow(r, _):
        for j in range(128 // 16):
            zbuf[r, pl.ds(16 * j, 16)] = zv
        return 0

    lax.fori_loop(0, zbuf.shape[0], zrow, 0)

    def zdrow(r, _):
        zdb[pl.ds(16 * r, 16)] = zv
        return 0

    lax.fori_loop(0, rows_per_sub // 16, zdrow, 0)
    zchunk = zbuf.shape[0]
    for i in range(rows_per_sub // zchunk):
        pltpu.sync_copy(zbuf, acc.at[pl.ds(sid * rows_per_sub
                                           + i * zchunk, zchunk)])
    pltpu.sync_copy(zdb, den.at[pl.ds(sid * rows_per_sub, rows_per_sub)])
    plsc.subcore_barrier()

    pltpu.sync_copy(mps, shv_v)
    shv = shv_v[...]
    pltpu.sync_copy(mpe, shv_v)
    shv = shv + shv_v[...]

    def lin_sp(g, b):
        base = e0 + g * _CHUNK
        pltpu.async_copy(src.at[pl.ds(base, _CHUNK)], srcv[b], lsp[b])
        pltpu.async_copy(pe.at[pl.ds(base, _CHUNK)], pev[b], lsp[b])

    def wait_sp(b):
        pltpu.make_async_copy(src.at[pl.ds(0, _CHUNK)], srcv[b], lsp[b]).wait()
        pltpu.make_async_copy(pe.at[pl.ds(0, _CHUNK)], pev[b], lsp[b]).wait()

    def lin_d(g, b):
        base = e0 + g * _CHUNK
        pltpu.async_copy(dst.at[pl.ds(base, _CHUNK)], dstv[b], ld[b])

    def wait_d(b):
        pltpu.make_async_copy(dst.at[pl.ds(0, _CHUNK)], dstv[b], ld[b]).wait()

    def gat(b):
        pltpu.async_copy(ft.at[srcv[b]], rows[b], gr[b])
        pltpu.async_copy(ps.at[srcv[b]], psg[b], gp[b])

    def wait_gat(b):
        pltpu.make_async_copy(ft.at[srcv[b]], rows[b], gr[b]).wait()
        pltpu.make_async_copy(ps.at[srcv[b]], psg[b], gp[b]).wait()

    def scat(b):
        pltpu.async_copy(rows[b], acc.at[dstv[b]], sr[b], add=True)
        pltpu.async_copy(wbuf[b], den.at[dstv[b]], sd[b], add=True)

    def wait_scat(b):
        pltpu.make_async_copy(rows[b], acc.at[dstv[b]], sr[b]).wait()
        pltpu.make_async_copy(wbuf[b], den.at[dstv[b]], sd[b]).wait()

    def scale(b):
        for gi in range(_CHUNK // 16):
            sl16 = pl.ds(16 * gi, 16)
            wv = jnp.exp(pev[b][sl16] + psg[b][sl16] - shv)
            wbuf[b][sl16] = wv
            for r in range(16):
                bc = _lane_bcast(wv, r)
                row = gi * 16 + r
                for j in range(128 // 16):
                    sl = pl.ds(16 * j, 16)
                    rows[b][row, sl] = rows[b][row, sl] * bc

    def step(g, b, first, last2):
        nb = 1 - b
        wait_gat(b)
        scale(b)
        wait_d(b)
        scat(b)
        wait_sp(nb)
        if not first:
            wait_scat(nb)
        lin_d(g + 1, nb)
        gat(nb)
        if last2 is None:
            @pl.when(g <= n_chunks - 3)
            def _():
                lin_sp(g + 2, b)
        elif not last2:
            lin_sp(g + 2, b)

    lin_sp(0, 0)
    lin_d(0, 0)
    lin_sp(1, 1)
    wait_sp(0)
    gat(0)
    step(0, 0, True, False)

    def pair(p, _):
        g = 2 * p + 1
        step(g, 1, False, None)
        step(g + 1, 0, False, None)
        return 0

    lax.fori_loop(0, (n_chunks - 4) // 2, pair, 0)
    step(n_chunks - 3, 1, False, False)
    step(n_chunks - 2, 0, False, True)

    wait_gat(1)
    scale(1)
    wait_d(1)
    scat(1)
    wait_scat(0)
    wait_scat(1)
    plsc.subcore_barrier()

    pltpu.sync_copy(
        acc.at[pl.ds(sid * rows_per_sub, rows_per_sub)],
        acc2.at[pl.ds(cid * n_rows + sid * rows_per_sub, rows_per_sub)])
    pltpu.sync_copy(
        den.at[pl.ds(sid * rows_per_sub, rows_per_sub)],
        den2.at[pl.ds(cid * n_rows + sid * rows_per_sub, rows_per_sub)])


def _edge_pallas(ft, src, dst, pe, ps, mps16, mpe16):
    n = ft.shape[0]
    mesh = plsc.VectorSubcoreMesh(core_axis_name="c", subcore_axis_name="s")
    return pl.kernel(
        _edge_body,
        out_type=[jax.ShapeDtypeStruct((2 * n, 128), jnp.float32),
                  jax.ShapeDtypeStruct((2 * n,), jnp.float32)],
        mesh=mesh,
        scratch_types=[
            pltpu.VMEM_SHARED((_NPAD, 128), jnp.float32),
            pltpu.VMEM_SHARED((_NPAD,), jnp.float32),
            pltpu.VMEM((_CHUNK,), jnp.int32),
            pltpu.VMEM((_CHUNK,), jnp.int32),
            pltpu.VMEM((_CHUNK,), jnp.float32),
            pltpu.VMEM((_CHUNK,), jnp.float32),
            pltpu.VMEM((_CHUNK,), jnp.float32),
            pltpu.VMEM((_CHUNK, 128), jnp.float32),
            pltpu.VMEM((_CHUNK,), jnp.int32),
            pltpu.VMEM((_CHUNK,), jnp.int32),
            pltpu.VMEM((_CHUNK,), jnp.float32),
            pltpu.VMEM((_CHUNK,), jnp.float32),
            pltpu.VMEM((_CHUNK,), jnp.float32),
            pltpu.VMEM((_CHUNK, 128), jnp.float32),
            pltpu.VMEM((32, 128), jnp.float32),
            pltpu.VMEM((640,), jnp.float32),
            pltpu.VMEM((16,), jnp.float32),
        ] + [pltpu.SemaphoreType.DMA] * 12,
    )(ft, src, dst, pe, ps, mps16, mpe16)


def kernel(x0, x1, edge_index0, edge_index1, eattr0, eattr1, Wgat0, Wgat1,
           a0, a1, Wih0, Whh0, bih0, bhh0, Wih1, Whh1, bih1, bhh1):
    N, F = x1.shape
    DE = eattr1.shape[1]
    epad = _EPAD - edge_index1.shape[1]
    spread = (jnp.arange(epad, dtype=jnp.int32) % 128)
    src = jnp.concatenate([edge_index1[0], spread * 64])
    dst = jnp.concatenate([edge_index1[1], N + 16 + spread])
    W0, W1 = _lstm_pallas(Wgat0, Wih0, Whh0, bih0, bhh0,
                          Wgat1, Wih1, Whh1, bih1, bhh1)
    xp = jnp.pad(x1, ((0, _NPAD - N), (0, 0)))
    pes, mpes, a_ss = [], [], []
    for a in (a0, a1):
        a_e = a[0, F:F + DE].reshape(DE, 1)
        pe, mpe = _pe_pallas(eattr1, a_e)
        pes.append(jnp.pad(pe.reshape(-1), (0, epad)))
        mpes.append(jnp.broadcast_to(mpe.reshape(1), (16,)))
        a_ss.append(a[0, :F].reshape(F, 1))
    ft, ps, mps = _ft_ps_pallas(xp, W0, a_ss[0])
    mps16 = jnp.broadcast_to(mps.reshape(1), (16,))
    acc2, den2 = _edge_pallas(ft, src, dst, pes[0], ps.reshape(-1),
                              mps16, mpes[0])
    ft, ps, mps = _norm_ft_ps_pallas(acc2, den2, W1, a_ss[1])
    mps16 = jnp.broadcast_to(mps.reshape(1), (16,))
    acc2, den2 = _edge_pallas(ft, src, dst, pes[1], ps.reshape(-1),
                              mps16, mpes[1])
    return _norm_pallas(acc2, den2)[:N]


# speedup vs baseline: 1.8307x; 1.0559x over previous
import functools

import jax
import jax.numpy as jnp
from jax import lax
from jax.experimental import pallas as pl
from jax.experimental.pallas import tpu as pltpu
from jax.experimental.pallas import tpu_sc as plsc

_NEG_SLOPE = 0.01
_TW = 144
_CHUNK = 80
_NPAD = 10240

_BCAST_DN = lax.GatherDimensionNumbers(
    offset_dims=(), collapsed_slice_dims=(0,), start_index_map=(0,))


def _lane_bcast(vec16, lane):
    idx = jnp.full((16, 1), lane, jnp.int32)
    return lax.gather(vec16, idx, _BCAST_DN, slice_sizes=(1,),
                      mode=lax.GatherScatterMode.PROMISE_IN_BOUNDS)


def _lstm_body(wg0, wihT0, whhT0, b0, wg1, wihT1, whhT1, b1,
               out0, out1, zx0, zx1):
    T, H = wg0.shape

    def one_pass(src0, src1):
        zx0[...] = jnp.dot(src0, wihT0[...],
                           preferred_element_type=jnp.float32) + b0[...]
        zx1[...] = jnp.dot(src1, wihT1[...],
                           preferred_element_type=jnp.float32) + b1[...]

        def step(t, carry):
            h0, c0, h1, c1 = carry

            def cell(zx, h, c, whhT, out):
                z = zx[pl.ds(t, 1), :] + jnp.dot(
                    h, whhT[...], preferred_element_type=jnp.float32)
                i = jax.nn.sigmoid(z[:, 0:H])
                f = jax.nn.sigmoid(z[:, H:2 * H])
                g = jnp.tanh(z[:, 2 * H:3 * H])
                o = jax.nn.sigmoid(z[:, 3 * H:4 * H])
                c = f * c + i * g
                h = o * jnp.tanh(c)
                out[pl.ds(t, 1), :] = h
                return h, c

            h0, c0 = cell(zx0, h0, c0, whhT0, out0)
            h1, c1 = cell(zx1, h1, c1, whhT1, out1)
            return h0, c0, h1, c1

        z = jnp.zeros((1, H), jnp.float32)
        lax.fori_loop(0, T, step, (z, z, z, z))

    one_pass(wg0[...], wg1[...])
    one_pass(out0[...], out1[...])


def _lstm_pallas(Wgat0, Wih0, Whh0, bih0, bhh0, Wgat1, Wih1, Whh1, bih1, bhh1):
    b0 = (bih0 + bhh0).reshape(1, -1)
    b1 = (bih1 + bhh1).reshape(1, -1)
    return pl.pallas_call(
        _lstm_body,
        out_shape=[jax.ShapeDtypeStruct(Wgat0.shape, jnp.float32)] * 2,
        scratch_shapes=[pltpu.VMEM((128, 512), jnp.float32)] * 2,
    )(Wgat0, Wih0.T, Whh0.T, b0, Wgat1, Wih1.T, Whh1.T, b1)


def _ft_ps_body(x, W, a_s, ft, ps, mx):
    xv = x[...]
    ft[...] = jnp.dot(xv, W[...], preferred_element_type=jnp.float32)
    p = jnp.dot(xv, a_s[...], preferred_element_type=jnp.float32)
    ps[...] = p
    mx[...] = jnp.full((1, 1), jnp.max(p))


def _ft_ps_pallas(x, W, a_s):
    return pl.pallas_call(
        _ft_ps_body,
        out_shape=[jax.ShapeDtypeStruct(x.shape, jnp.float32),
                   jax.ShapeDtypeStruct((x.shape[0], 1), jnp.float32),
                   jax.ShapeDtypeStruct((1, 1), jnp.float32)],
    )(x, W, a_s)


def _norm_ft_ps_body(am, bm, da, db, W, a_s, ft, ps, mx):
    x = (am[...] + bm[...]) / (da[...] + db[...] + 1e-16)
    x = jnp.maximum(x, x * _NEG_SLOPE)
    ft[...] = jnp.dot(x, W[...], preferred_element_type=jnp.float32)
    p = jnp.dot(x, a_s[...], preferred_element_type=jnp.float32)
    ps[...] = p
    mx[...] = jnp.full((1, 1), jnp.max(p))


def _norm_ft_ps_pallas(acc2, den2, W, a_s):
    n = acc2.shape[0] // 2
    am, bm = acc2[:n], acc2[n:]
    da, db = den2[:n].reshape(n, 1), den2[n:].reshape(n, 1)
    return pl.pallas_call(
        _norm_ft_ps_body,
        out_shape=[jax.ShapeDtypeStruct((n, 128), jnp.float32),
                   jax.ShapeDtypeStruct((n, 1), jnp.float32),
                   jax.ShapeDtypeStruct((1, 1), jnp.float32)],
    )(am, bm, da, db, W, a_s)


def _norm_body(am, bm, da, db, out):
    x = (am[...] + bm[...]) / (da[...] + db[...] + 1e-16)
    out[...] = jnp.maximum(x, x * _NEG_SLOPE)


def _norm_pallas(acc2, den2):
    n = acc2.shape[0] // 2
    am, bm = acc2[:n], acc2[n:]
    da, db = den2[:n].reshape(n, 1), den2[n:].reshape(n, 1)
    return pl.pallas_call(
        _norm_body,
        out_shape=jax.ShapeDtypeStruct((n, 128), jnp.float32),
    )(am, bm, da, db)


def _pe_body(ea, a_e, pe, mx, mscr):
    i = pl.program_id(0)
    p = jnp.dot(ea[...], a_e[...], preferred_element_type=jnp.float32)
    pe[...] = p
    m = jnp.max(p)

    @pl.when(i == 0)
    def _():
        mscr[0, 0] = m

    @pl.when(i > 0)
    def _():
        mscr[0, 0] = jnp.maximum(mscr[0, 0], m)

    @pl.when(i == pl.num_programs(0) - 1)
    def _():
        mx[...] = jnp.full((1, 1), mscr[0, 0])


def _pe_pallas(eattr, a_e):
    E = eattr.shape[0]
    nb = 32
    blk = E // nb
    return pl.pallas_call(
        _pe_body,
        grid=(nb,),
        in_specs=[pl.BlockSpec((blk, eattr.shape[1]), lambda i: (i, 0)),
                  pl.BlockSpec((eattr.shape[1], 1), lambda i: (0, 0))],
        out_specs=[pl.BlockSpec((blk, 1), lambda i: (i, 0)),
                   pl.BlockSpec((1, 1), lambda i: (0, 0))],
        out_shape=[jax.ShapeDtypeStruct((E, 1), jnp.float32),
                   jax.ShapeDtypeStruct((1, 1), jnp.float32)],
        scratch_shapes=[pltpu.SMEM((1, 1), jnp.float32)],
    )(eattr, a_e)


def _edge_body(ft, src, dst, pe, ps, mps, mpe, acc2, den2,
               acc, den,
               srcv0, dstv0, pev0, psg0, wbuf0, rows0,
               srcv1, dstv1, pev1, psg1, wbuf1, rows1,
               zbuf, zdb, shv_v,
               lsp0, lsp1, ld0, ld1, gr0, gr1, gp0, gp1, sr0, sr1, sd0, sd1):
    info = plsc.get_sparse_core_info()
    nc, ns = info.num_cores, info.num_subcores
    cid = lax.axis_index("c")
    sid = lax.axis_index("s")
    wid = sid * nc + cid
    E = src.shape[0]
    n_rows = acc2.shape[0] // 2
    rows_per_sub = n_rows // ns
    edges_per_w = E // (nc * ns)
    n_chunks = edges_per_w // _CHUNK
    e0 = wid * edges_per_w

    srcv = (srcv0, srcv1)
    dstv = (dstv0, dstv1)
    pev = (pev0, pev1)
    psg = (psg0, psg1)
    wbuf = (wbuf0, wbuf1)
    rows = (rows0, rows1)
    lsp = (lsp0, lsp1)
    ld = (ld0, ld1)
    gr = (gr0, gr1)
    gp = (gp0, gp1)
    sr = (sr0, sr1)
    sd = (sd0, sd1)

    zv = jnp.zeros((16,), jnp.float32)

    def zrow(r, _):
        for j in range(128 // 16):
            zbuf[r, pl.ds(16 * j, 16)] = zv
        return 0

    lax.fori_loop(0, zbuf.shape[0], zrow, 0)

    def zdrow(r, _):
        zdb[pl.ds(16 * r, 16)] = zv
        return 0

    lax.fori_loop(0, rows_per_sub // 16, zdrow, 0)
    zchunk = zbuf.shape[0]
    for i in range(rows_per_sub // zchunk):
        pltpu.sync_copy(zbuf, acc.at[pl.ds(sid * rows_per_sub
                                           + i * zchunk, zchunk)])
    pltpu.sync_copy(zdb, den.at[pl.ds(sid * rows_per_sub, rows_per_sub)])
    plsc.subcore_barrier()

    pltpu.sync_copy(mps, shv_v)
    shv = shv_v[...]
    pltpu.sync_copy(mpe, shv_v)
    shv = shv + shv_v[...]

    def lin_sp(g, b):
        base = e0 + g * _CHUNK
        pltpu.async_copy(src.at[pl.ds(base, _CHUNK)], srcv[b], lsp[b])
        pltpu.async_copy(pe.at[pl.ds(base, _CHUNK)], pev[b], lsp[b])

    def wait_sp(b):
        pltpu.make_async_copy(src.at[pl.ds(0, _CHUNK)], srcv[b], lsp[b]).wait()
        pltpu.make_async_copy(pe.at[pl.ds(0, _CHUNK)], pev[b], lsp[b]).wait()

    def lin_d(g, b):
        base = e0 + g * _CHUNK
        pltpu.async_copy(dst.at[pl.ds(base, _CHUNK)], dstv[b], ld[b])

    def wait_d(b):
        pltpu.make_async_copy(dst.at[pl.ds(0, _CHUNK)], dstv[b], ld[b]).wait()

    def gat(b):
        pltpu.async_copy(ft.at[srcv[b]], rows[b], gr[b])
        pltpu.async_copy(ps.at[srcv[b]], psg[b], gp[b])

    def wait_gat(b):
        pltpu.make_async_copy(ft.at[srcv[b]], rows[b], gr[b]).wait()
        pltpu.make_async_copy(ps.at[srcv[b]], psg[b], gp[b]).wait()

    def scat(b):
        pltpu.async_copy(rows[b], acc.at[dstv[b]], sr[b], add=True)
        pltpu.async_copy(wbuf[b], den.at[dstv[b]], sd[b], add=True)

    def wait_scat(b):
        pltpu.make_async_copy(rows[b], acc.at[dstv[b]], sr[b]).wait()
        pltpu.make_async_copy(wbuf[b], den.at[dstv[b]], sd[b]).wait()

    def scale(b):
        for gi in range(_CHUNK // 16):
            sl16 = pl.ds(16 * gi, 16)
            wv = jnp.exp(pev[b][sl16] + psg[b][sl16] - shv)
            wbuf[b][sl16] = wv
            for r in range(16):
                bc = _lane_bcast(wv, r)
                row = gi * 16 + r
                for j in range(128 // 16):
                    sl = pl.ds(16 * j, 16)
                    rows[b][row, sl] = rows[b][row, sl] * bc

    def step(g, b, first, last2):
        nb = 1 - b
        wait_gat(b)
        scale(b)
        wait_d(b)
        scat(b)
        wait_sp(nb)
        if not first:
            wait_scat(nb)
        lin_d(g + 1, nb)
        gat(nb)
        if last2 is None:
            @pl.when(g <= n_chunks - 3)
            def _():
                lin_sp(g + 2, b)
        elif not last2:
            lin_sp(g + 2, b)

    lin_sp(0, 0)
    lin_d(0, 0)
    lin_sp(1, 1)
    wait_sp(0)
    gat(0)
    step(0, 0, True, False)

    def pair(p, _):
        g = 2 * p + 1
        step(g, 1, False, None)
        step(g + 1, 0, False, None)
        return 0

    lax.fori_loop(0, (n_chunks - 3) // 2, pair, 0)
    step(n_chunks - 2, 1, False, True)

    wait_gat(0)
    scale(0)
    wait_d(0)
    scat(0)
    wait_scat(1)
    wait_scat(0)
    plsc.subcore_barrier()

    pltpu.sync_copy(
        acc.at[pl.ds(sid * rows_per_sub, rows_per_sub)],
        acc2.at[pl.ds(cid * n_rows + sid * rows_per_sub, rows_per_sub)])
    pltpu.sync_copy(
        den.at[pl.ds(sid * rows_per_sub, rows_per_sub)],
        den2.at[pl.ds(cid * n_rows + sid * rows_per_sub, rows_per_sub)])


def _edge_pallas(ft, src, dst, pe, ps, mps16, mpe16):
    n = ft.shape[0]
    mesh = plsc.VectorSubcoreMesh(core_axis_name="c", subcore_axis_name="s")
    return pl.kernel(
        _edge_body,
        out_type=[jax.ShapeDtypeStruct((2 * n, 128), jnp.float32),
                  jax.ShapeDtypeStruct((2 * n,), jnp.float32)],
        mesh=mesh,
        scratch_types=[
            pltpu.VMEM_SHARED((_NPAD, 128), jnp.float32),
            pltpu.VMEM_SHARED((_NPAD,), jnp.float32),
            pltpu.VMEM((_CHUNK,), jnp.int32),
            pltpu.VMEM((_CHUNK,), jnp.int32),
            pltpu.VMEM((_CHUNK,), jnp.float32),
            pltpu.VMEM((_CHUNK,), jnp.float32),
            pltpu.VMEM((_CHUNK,), jnp.float32),
            pltpu.VMEM((_CHUNK, 128), jnp.float32),
            pltpu.VMEM((_CHUNK,), jnp.int32),
            pltpu.VMEM((_CHUNK,), jnp.int32),
            pltpu.VMEM((_CHUNK,), jnp.float32),
            pltpu.VMEM((_CHUNK,), jnp.float32),
            pltpu.VMEM((_CHUNK,), jnp.float32),
            pltpu.VMEM((_CHUNK, 128), jnp.float32),
            pltpu.VMEM((32, 128), jnp.float32),
            pltpu.VMEM((640,), jnp.float32),
            pltpu.VMEM((16,), jnp.float32),
        ] + [pltpu.SemaphoreType.DMA] * 12,
    )(ft, src, dst, pe, ps, mps16, mpe16)


def kernel(x0, x1, edge_index0, edge_index1, eattr0, eattr1, Wgat0, Wgat1,
           a0, a1, Wih0, Whh0, bih0, bhh0, Wih1, Whh1, bih1, bhh1):
    N, F = x1.shape
    DE = eattr1.shape[1]
    src = edge_index1[0]
    dst = edge_index1[1]
    W0, W1 = _lstm_pallas(Wgat0, Wih0, Whh0, bih0, bhh0,
                          Wgat1, Wih1, Whh1, bih1, bhh1)
    xp = jnp.pad(x1, ((0, _NPAD - N), (0, 0)))
    pes, mpes, a_ss = [], [], []
    for a in (a0, a1):
        a_e = a[0, F:F + DE].reshape(DE, 1)
        pe, mpe = _pe_pallas(eattr1, a_e)
        pes.append(pe.reshape(-1))
        mpes.append(jnp.broadcast_to(mpe.reshape(1), (16,)))
        a_ss.append(a[0, :F].reshape(F, 1))
    ft, ps, mps = _ft_ps_pallas(xp, W0, a_ss[0])
    mps16 = jnp.broadcast_to(mps.reshape(1), (16,))
    acc2, den2 = _edge_pallas(ft, src, dst, pes[0], ps.reshape(-1),
                              mps16, mpes[0])
    ft, ps, mps = _norm_ft_ps_pallas(acc2, den2, W1, a_ss[1])
    mps16 = jnp.broadcast_to(mps.reshape(1), (16,))
    acc2, den2 = _edge_pallas(ft, src, dst, pes[1], ps.reshape(-1),
                              mps16, mpes[1])
    return _norm_pallas(acc2, den2)[:N]


# speedup vs baseline: 1.9301x; 1.0543x over previous
import functools

import jax
import jax.numpy as jnp
from jax import lax
from jax.experimental import pallas as pl
from jax.experimental.pallas import tpu as pltpu
from jax.experimental.pallas import tpu_sc as plsc

_NEG_SLOPE = 0.01
_TW = 144
_CHUNK = 80
_NPAD = 10240

_BCAST_DN = lax.GatherDimensionNumbers(
    offset_dims=(), collapsed_slice_dims=(0,), start_index_map=(0,))


def _lane_bcast(vec16, lane):
    idx = jnp.full((16, 1), lane, jnp.int32)
    return lax.gather(vec16, idx, _BCAST_DN, slice_sizes=(1,),
                      mode=lax.GatherScatterMode.PROMISE_IN_BOUNDS)


def _lstm_body(wg0, wihT0, whhT0, b0, wg1, wihT1, whhT1, b1,
               out0, out1, zx0, zx1):
    T, H = wg0.shape

    def one_pass(src0, src1):
        zx0[...] = jnp.dot(src0, wihT0[...],
                           preferred_element_type=jnp.float32) + b0[...]
        zx1[...] = jnp.dot(src1, wihT1[...],
                           preferred_element_type=jnp.float32) + b1[...]

        def step(t, carry):
            h0, c0, h1, c1 = carry

            def cell(zx, h, c, whhT, out):
                z = zx[pl.ds(t, 1), :] + jnp.dot(
                    h, whhT[...], preferred_element_type=jnp.float32)
                i = jax.nn.sigmoid(z[:, 0:H])
                f = jax.nn.sigmoid(z[:, H:2 * H])
                g = jnp.tanh(z[:, 2 * H:3 * H])
                o = jax.nn.sigmoid(z[:, 3 * H:4 * H])
                c = f * c + i * g
                h = o * jnp.tanh(c)
                out[pl.ds(t, 1), :] = h
                return h, c

            h0, c0 = cell(zx0, h0, c0, whhT0, out0)
            h1, c1 = cell(zx1, h1, c1, whhT1, out1)
            return h0, c0, h1, c1

        z = jnp.zeros((1, H), jnp.float32)
        lax.fori_loop(0, T, step, (z, z, z, z))

    one_pass(wg0[...], wg1[...])
    one_pass(out0[...], out1[...])


def _lstm_pallas(Wgat0, Wih0, Whh0, bih0, bhh0, Wgat1, Wih1, Whh1, bih1, bhh1):
    b0 = (bih0 + bhh0).reshape(1, -1)
    b1 = (bih1 + bhh1).reshape(1, -1)
    return pl.pallas_call(
        _lstm_body,
        out_shape=[jax.ShapeDtypeStruct(Wgat0.shape, jnp.float32)] * 2,
        scratch_shapes=[pltpu.VMEM((128, 512), jnp.float32)] * 2,
    )(Wgat0, Wih0.T, Whh0.T, b0, Wgat1, Wih1.T, Whh1.T, b1)


def _ft_ps_body(x, W, a_s, ft, ps, mx):
    n = x.shape[0]
    xv = x[...]
    ft[pl.ds(0, n), :] = jnp.dot(xv, W[...], preferred_element_type=jnp.float32)
    p = jnp.dot(xv, a_s[...], preferred_element_type=jnp.float32)
    ps[pl.ds(0, n), :] = p
    mx[...] = jnp.full((1, 1), jnp.max(p))


def _ft_ps_pallas(x, W, a_s):
    return pl.pallas_call(
        _ft_ps_body,
        out_shape=[jax.ShapeDtypeStruct((_NPAD, x.shape[1]), jnp.float32),
                   jax.ShapeDtypeStruct((_NPAD, 1), jnp.float32),
                   jax.ShapeDtypeStruct((1, 1), jnp.float32)],
    )(x, W, a_s)


def _norm_x(acc2, den2, nreal):
    am = acc2[pl.ds(0, nreal), :]
    bm = acc2[pl.ds(_NPAD, nreal), :]
    da = den2[pl.ds(0, nreal), :]
    db = den2[pl.ds(_NPAD, nreal), :]
    x = (am + bm) / (da + db + 1e-16)
    return jnp.maximum(x, x * _NEG_SLOPE)


def _norm_ft_ps_body(acc2, den2, W, a_s, ft, ps, mx):
    n = ft.shape[0] - (_NPAD - 10000)
    x = _norm_x(acc2, den2, 10000)
    ft[pl.ds(0, 10000), :] = jnp.dot(x, W[...],
                                     preferred_element_type=jnp.float32)
    p = jnp.dot(x, a_s[...], preferred_element_type=jnp.float32)
    ps[pl.ds(0, 10000), :] = p
    mx[...] = jnp.full((1, 1), jnp.max(p))


def _norm_ft_ps_pallas(acc2, den2, W, a_s):
    return pl.pallas_call(
        _norm_ft_ps_body,
        out_shape=[jax.ShapeDtypeStruct((_NPAD, 128), jnp.float32),
                   jax.ShapeDtypeStruct((_NPAD, 1), jnp.float32),
                   jax.ShapeDtypeStruct((1, 1), jnp.float32)],
    )(acc2, den2.reshape(-1, 1), W, a_s)


def _norm_body(acc2, den2, out):
    out[...] = _norm_x(acc2, den2, out.shape[0])


def _norm_pallas(acc2, den2, nreal):
    return pl.pallas_call(
        _norm_body,
        out_shape=jax.ShapeDtypeStruct((nreal, 128), jnp.float32),
    )(acc2, den2.reshape(-1, 1))


def _pe_body(ea, a_e0, a_e1, pe0, pe1, mx, mscr):
    i = pl.program_id(0)
    eav = ea[...]
    p0 = jnp.dot(eav, a_e0[...], preferred_element_type=jnp.float32)
    p1 = jnp.dot(eav, a_e1[...], preferred_element_type=jnp.float32)
    pe0[...] = p0
    pe1[...] = p1
    m0 = jnp.max(p0)
    m1 = jnp.max(p1)

    @pl.when(i == 0)
    def _():
        mscr[0, 0] = m0
        mscr[0, 1] = m1

    @pl.when(i > 0)
    def _():
        mscr[0, 0] = jnp.maximum(mscr[0, 0], m0)
        mscr[0, 1] = jnp.maximum(mscr[0, 1], m1)

    @pl.when(i == pl.num_programs(0) - 1)
    def _():
        mx[...] = jnp.concatenate(
            [jnp.full((1, 1), mscr[0, 0]), jnp.full((1, 1), mscr[0, 1])],
            axis=1)


def _pe_pallas(eattr, a_e0, a_e1):
    E = eattr.shape[0]
    nb = 32
    blk = E // nb
    de = eattr.shape[1]
    return pl.pallas_call(
        _pe_body,
        grid=(nb,),
        in_specs=[pl.BlockSpec((blk, de), lambda i: (i, 0)),
                  pl.BlockSpec((de, 1), lambda i: (0, 0)),
                  pl.BlockSpec((de, 1), lambda i: (0, 0))],
        out_specs=[pl.BlockSpec((blk, 1), lambda i: (i, 0)),
                   pl.BlockSpec((blk, 1), lambda i: (i, 0)),
                   pl.BlockSpec((1, 2), lambda i: (0, 0))],
        out_shape=[jax.ShapeDtypeStruct((E, 1), jnp.float32),
                   jax.ShapeDtypeStruct((E, 1), jnp.float32),
                   jax.ShapeDtypeStruct((1, 2), jnp.float32)],
        scratch_shapes=[pltpu.SMEM((1, 2), jnp.float32)],
    )(eattr, a_e0, a_e1)


def _edge_body(ft, src, dst, pe, ps, mps, mpe, acc2, den2,
               acc, den,
               srcv0, dstv0, pev0, psg0, wbuf0, rows0,
               srcv1, dstv1, pev1, psg1, wbuf1, rows1,
               zbuf, zdb, shv_v,
               lsp0, lsp1, ld0, ld1, gr0, gr1, gp0, gp1, sr0, sr1, sd0, sd1):
    info = plsc.get_sparse_core_info()
    nc, ns = info.num_cores, info.num_subcores
    cid = lax.axis_index("c")
    sid = lax.axis_index("s")
    wid = sid * nc + cid
    E = src.shape[0]
    n_rows = acc2.shape[0] // 2
    rows_per_sub = n_rows // ns
    edges_per_w = E // (nc * ns)
    n_chunks = edges_per_w // _CHUNK
    e0 = wid * edges_per_w

    srcv = (srcv0, srcv1)
    dstv = (dstv0, dstv1)
    pev = (pev0, pev1)
    psg = (psg0, psg1)
    wbuf = (wbuf0, wbuf1)
    rows = (rows0, rows1)
    lsp = (lsp0, lsp1)
    ld = (ld0, ld1)
    gr = (gr0, gr1)
    gp = (gp0, gp1)
    sr = (sr0, sr1)
    sd = (sd0, sd1)

    zv = jnp.zeros((16,), jnp.float32)

    def zrow(r, _):
        for j in range(128 // 16):
            zbuf[r, pl.ds(16 * j, 16)] = zv
        return 0

    lax.fori_loop(0, zbuf.shape[0], zrow, 0)

    def zdrow(r, _):
        zdb[pl.ds(16 * r, 16)] = zv
        return 0

    lax.fori_loop(0, rows_per_sub // 16, zdrow, 0)
    zchunk = zbuf.shape[0]
    for i in range(rows_per_sub // zchunk):
        pltpu.sync_copy(zbuf, acc.at[pl.ds(sid * rows_per_sub
                                           + i * zchunk, zchunk)])
    pltpu.sync_copy(zdb, den.at[pl.ds(sid * rows_per_sub, rows_per_sub)])
    plsc.subcore_barrier()

    pltpu.sync_copy(mps, shv_v)
    shv = shv_v[...]
    pltpu.sync_copy(mpe, shv_v)
    shv = shv + shv_v[...]

    def lin_sp(g, b):
        base = e0 + g * _CHUNK
        pltpu.async_copy(src.at[pl.ds(base, _CHUNK)], srcv[b], lsp[b])
        pltpu.async_copy(pe.at[pl.ds(base, _CHUNK)], pev[b], lsp[b])

    def wait_sp(b):
        pltpu.make_async_copy(src.at[pl.ds(0, _CHUNK)], srcv[b], lsp[b]).wait()
        pltpu.make_async_copy(pe.at[pl.ds(0, _CHUNK)], pev[b], lsp[b]).wait()

    def lin_d(g, b):
        base = e0 + g * _CHUNK
        pltpu.async_copy(dst.at[pl.ds(base, _CHUNK)], dstv[b], ld[b])

    def wait_d(b):
        pltpu.make_async_copy(dst.at[pl.ds(0, _CHUNK)], dstv[b], ld[b]).wait()

    def gat(b):
        pltpu.async_copy(ft.at[srcv[b]], rows[b], gr[b])
        pltpu.async_copy(ps.at[srcv[b]], psg[b], gp[b])

    def wait_gat(b):
        pltpu.make_async_copy(ft.at[srcv[b]], rows[b], gr[b]).wait()
        pltpu.make_async_copy(ps.at[srcv[b]], psg[b], gp[b]).wait()

    def scat(b):
        pltpu.async_copy(rows[b], acc.at[dstv[b]], sr[b], add=True)
        pltpu.async_copy(wbuf[b], den.at[dstv[b]], sd[b], add=True)

    def wait_scat(b):
        pltpu.make_async_copy(rows[b], acc.at[dstv[b]], sr[b]).wait()
        pltpu.make_async_copy(wbuf[b], den.at[dstv[b]], sd[b]).wait()

    def scale(b):
        for gi in range(_CHUNK // 16):
            sl16 = pl.ds(16 * gi, 16)
            wv = jnp.exp(pev[b][sl16] + psg[b][sl16] - shv)
            wbuf[b][sl16] = wv
            for r in range(16):
                bc = _lane_bcast(wv, r)
                row = gi * 16 + r
                for j in range(128 // 16):
                    sl = pl.ds(16 * j, 16)
                    rows[b][row, sl] = rows[b][row, sl] * bc

    def step(g, b, first, last2):
        nb = 1 - b
        wait_gat(b)
        scale(b)
        wait_d(b)
        scat(b)
        wait_sp(nb)
        if not first:
            wait_scat(nb)
        lin_d(g + 1, nb)
        gat(nb)
        if last2 is None:
            @pl.when(g <= n_chunks - 3)
            def _():
                lin_sp(g + 2, b)
        elif not last2:
            lin_sp(g + 2, b)

    lin_sp(0, 0)
    lin_d(0, 0)
    lin_sp(1, 1)
    wait_sp(0)
    gat(0)
    step(0, 0, True, False)

    def pair(p, _):
        g = 2 * p + 1
        step(g, 1, False, None)
        step(g + 1, 0, False, None)
        return 0

    lax.fori_loop(0, (n_chunks - 3) // 2, pair, 0)
    step(n_chunks - 2, 1, False, True)

    wait_gat(0)
    scale(0)
    wait_d(0)
    scat(0)
    wait_scat(1)
    wait_scat(0)
    plsc.subcore_barrier()

    pltpu.sync_copy(
        acc.at[pl.ds(sid * rows_per_sub, rows_per_sub)],
        acc2.at[pl.ds(cid * n_rows + sid * rows_per_sub, rows_per_sub)])
    pltpu.sync_copy(
        den.at[pl.ds(sid * rows_per_sub, rows_per_sub)],
        den2.at[pl.ds(cid * n_rows + sid * rows_per_sub, rows_per_sub)])


def _edge_pallas(ft, src, dst, pe, ps, mps16, mpe16):
    n = ft.shape[0]
    mesh = plsc.VectorSubcoreMesh(core_axis_name="c", subcore_axis_name="s")
    return pl.kernel(
        _edge_body,
        out_type=[jax.ShapeDtypeStruct((2 * n, 128), jnp.float32),
                  jax.ShapeDtypeStruct((2 * n,), jnp.float32)],
        mesh=mesh,
        scratch_types=[
            pltpu.VMEM_SHARED((_NPAD, 128), jnp.float32),
            pltpu.VMEM_SHARED((_NPAD,), jnp.float32),
            pltpu.VMEM((_CHUNK,), jnp.int32),
            pltpu.VMEM((_CHUNK,), jnp.int32),
            pltpu.VMEM((_CHUNK,), jnp.float32),
            pltpu.VMEM((_CHUNK,), jnp.float32),
            pltpu.VMEM((_CHUNK,), jnp.float32),
            pltpu.VMEM((_CHUNK, 128), jnp.float32),
            pltpu.VMEM((_CHUNK,), jnp.int32),
            pltpu.VMEM((_CHUNK,), jnp.int32),
            pltpu.VMEM((_CHUNK,), jnp.float32),
            pltpu.VMEM((_CHUNK,), jnp.float32),
            pltpu.VMEM((_CHUNK,), jnp.float32),
            pltpu.VMEM((_CHUNK, 128), jnp.float32),
            pltpu.VMEM((32, 128), jnp.float32),
            pltpu.VMEM((640,), jnp.float32),
            pltpu.VMEM((16,), jnp.float32),
        ] + [pltpu.SemaphoreType.DMA] * 12,
    )(ft, src, dst, pe, ps, mps16, mpe16)


def kernel(x0, x1, edge_index0, edge_index1, eattr0, eattr1, Wgat0, Wgat1,
           a0, a1, Wih0, Whh0, bih0, bhh0, Wih1, Whh1, bih1, bhh1):
    N, F = x1.shape
    DE = eattr1.shape[1]
    src = edge_index1[0]
    dst = edge_index1[1]
    W0, W1 = _lstm_pallas(Wgat0, Wih0, Whh0, bih0, bhh0,
                          Wgat1, Wih1, Whh1, bih1, bhh1)
    pe0, pe1, mpe = _pe_pallas(eattr1,
                               a0[0, F:F + DE].reshape(DE, 1),
                               a1[0, F:F + DE].reshape(DE, 1))
    mpe0_16 = jnp.broadcast_to(mpe[0:1, 0], (16,))
    mpe1_16 = jnp.broadcast_to(mpe[0:1, 1], (16,))
    ft, ps, mps = _ft_ps_pallas(x1, W0, a0[0, :F].reshape(F, 1))
    mps16 = jnp.broadcast_to(mps.reshape(1), (16,))
    acc2, den2 = _edge_pallas(ft, src, dst, pe0.reshape(-1), ps.reshape(-1),
                              mps16, mpe0_16)
    ft, ps, mps = _norm_ft_ps_pallas(acc2, den2, W1, a1[0, :F].reshape(F, 1))
    mps16 = jnp.broadcast_to(mps.reshape(1), (16,))
    acc2, den2 = _edge_pallas(ft, src, dst, pe1.reshape(-1), ps.reshape(-1),
                              mps16, mpe1_16)
    return _norm_pallas(acc2, den2, N)
